# baseline (speedup 1.0000x reference)
import functools

import jax
import jax.numpy as jnp
from jax import lax
from jax.experimental import pallas as pl
from jax.experimental.pallas import tpu as pltpu
from jax.experimental.pallas import tpu_sc as plsc

_B = 4096
_S = 200
_D = 64
_R = 4
_OUT_S = _R + _S
_PS = 112
_NW = 32
_LANES = 16
_C0 = 128
_C1 = _S - _C0


def _make_sc_kernel(nb):
    mesh = plsc.VectorSubcoreMesh(core_axis_name="c", subcore_axis_name="s")
    seq_per_w = nb // _NW

    @functools.partial(
        pl.kernel,
        mesh=mesh,
        compiler_params=pltpu.CompilerParams(use_tc_tiling_on_sc=False),
        out_type=jax.ShapeDtypeStruct((nb, 2 * _PS, _D // 2), jnp.float32),
        scratch_types=[
            pltpu.VMEM((256,), jnp.int32),
            pltpu.VMEM((256,), jnp.int32),
            pltpu.VMEM((_S, _D // 2), jnp.float32),
            pltpu.VMEM((_S, _D // 2), jnp.float32),
            pltpu.SemaphoreType.DMA,
            pltpu.SemaphoreType.DMA,
            pltpu.SemaphoreType.DMA,
            pltpu.SemaphoreType.DMA,
            pltpu.SemaphoreType.DMA,
            pltpu.SemaphoreType.DMA,
        ],
    )
    def emb_kernel(tok_hbm, table_hbm, out_hbm,
                   idx0, idx1, gat0, gat1,
                   gsem0, gsem1, osem0, osem1, isem0, isem1):
        wid = lax.axis_index("s") * 2 + lax.axis_index("c")
        base = wid * seq_per_w

        def start_idx(i, idx, isem):
            pltpu.async_copy(tok_hbm.at[pl.ds((base + i) * 256, 256)], idx,
                             isem)

        def drain_idx(idx, isem):
            pltpu.make_async_copy(tok_hbm.at[pl.ds(0, 256)], idx, isem).wait()

        def start_gather(idx, gat, gsem):
            pltpu.async_copy(table_hbm.at[idx.at[pl.ds(0, _C0)]],
                             gat.at[pl.ds(0, _C0)], gsem)
            pltpu.async_copy(table_hbm.at[idx.at[pl.ds(_C0, _C1)]],
                             gat.at[pl.ds(_C0, _C1)], gsem)

        def drain_gather(gat, gsem):
            pltpu.make_async_copy(table_hbm.at[pl.ds(0, _S)], gat, gsem).wait()

        def drain_out(gat, b, osem):
            pltpu.make_async_copy(gat, out_hbm.at[b, pl.ds(0, _S)],
                                  osem).wait()

        start_idx(0, idx0, isem0)
        start_idx(1, idx1, isem1)
        drain_idx(idx0, isem0)
        drain_idx(idx1, isem1)
        start_gather(idx0, gat0, gsem0)
        start_gather(idx1, gat1, gsem1)

        def seq_body(j, carry):
            for s, (idx, gat, gsem, osem, isem) in enumerate((
                    (idx0, gat0, gsem0, osem0, isem0),
                    (idx1, gat1, gsem1, osem1, isem1))):
                i = 2 * j + s
                b = base + i
                drain_gather(gat, gsem)

                @pl.when(i + 2 < seq_per_w)
                def _():
                    start_idx(i + 2, idx, isem)

                pltpu.async_copy(gat, out_hbm.at[b, pl.ds(0, _S)], osem)

                @pl.when(i + 2 < seq_per_w)
                def _():
                    drain_idx(idx, isem)
                    drain_out(gat, b, osem)
                    start_gather(idx, gat, gsem)
            return carry

        lax.fori_loop(0, seq_per_w // 2, seq_body, 0)
        drain_out(gat0, 0, osem0)
        drain_out(gat1, 0, osem1)

    return emb_kernel


_NHALF = _B // 2
_EMB_KERNEL = _make_sc_kernel(_NHALF)

_BBLK = 128
_HBLKS = _NHALF // _BBLK
_PL = _PS // 2
_HD = _D // 2


def _tc_body(x_ref, pos_ref, reg_ref, o_ref):
    w = lax.bitcast_convert_type(x_ref[...], jnp.int32)
    ev = lax.bitcast_convert_type(w << 16, jnp.float32)
    od = lax.bitcast_convert_type(w & jnp.int32(-65536), jnp.float32)
    pos = pos_ref[...]
    for half, part in ((0, ev), (1, od)):
        z = lax.transpose(part, (1, 2, 0))
        z = z.reshape(2 * _PS, _HD, _BBLK)[: _S]
        o_ref[pl.ds(_R, _S), pl.ds(half * _HD, _HD)] = (
            z + pos[:, half * _HD:(half + 1) * _HD][:, :, None])
    o_ref[pl.ds(0, _R)] = jnp.broadcast_to(
        reg_ref[...][:, :, None], (_R, _D, _BBLK))


def _tc_transpose_body2(prev_ref, x_ref, pos_ref, reg_ref, o_ref):
    del prev_ref
    _tc_body(x_ref, pos_ref, reg_ref, o_ref)


_TC_STAGE_H1 = pl.pallas_call(
    _tc_body,
    grid=(_HBLKS,),
    in_specs=[
        pl.BlockSpec((_BBLK, _PL, 2 * _D), lambda i: (i, 0, 0)),
        pl.BlockSpec((_S, _D), lambda i: (0, 0)),
        pl.BlockSpec((_R, _D), lambda i: (0, 0)),
    ],
    out_specs=pl.BlockSpec((_OUT_S, _D, _BBLK), lambda i: (0, 0, i)),
    out_shape=jax.ShapeDtypeStruct((_OUT_S, _D, _B), jnp.float32),
)

_TC_STAGE_H2 = pl.pallas_call(
    _tc_transpose_body2,
    grid=(_HBLKS,),
    in_specs=[
        pl.BlockSpec(memory_space=pl.ANY),
        pl.BlockSpec((_BBLK, _PL, 2 * _D), lambda i: (i, 0, 0)),
        pl.BlockSpec((_S, _D), lambda i: (0, 0)),
        pl.BlockSpec((_R, _D), lambda i: (0, 0)),
    ],
    out_specs=pl.BlockSpec((_OUT_S, _D, _BBLK), lambda i: (0, 0, i + _HBLKS)),
    out_shape=jax.ShapeDtypeStruct((_OUT_S, _D, _B), jnp.float32),
    input_output_aliases={0: 0},
)


@jax.jit
def kernel(tokens, input_embedding, position_embedding, register_tokens):
    v = input_embedding.shape[0]
    tb16 = input_embedding.astype(jnp.bfloat16)
    tperm = tb16.reshape(v, 2, _HD).transpose(0, 2, 1)
    tpack = lax.bitcast_convert_type(tperm, jnp.float32)
    tok_flat = jnp.pad(tokens, ((0, 0), (0, 256 - _S))).reshape(-1)
    h1 = _EMB_KERNEL(tok_flat[: _NHALF * 256], tpack)
    h2 = _EMB_KERNEL(tok_flat[_NHALF * 256:], tpack)
    s1 = h1.reshape(_NHALF, _PL, 2 * _D)
    s2 = h2.reshape(_NHALF, _PL, 2 * _D)
    out_t = _TC_STAGE_H1(s1, position_embedding, register_tokens)
    out_t = _TC_STAGE_H2(out_t, s2, position_embedding, register_tokens)
    return out_t.transpose(2, 0, 1)


# speedup vs baseline: 1.9413x; 1.9413x over previous
import functools

import jax
import jax.numpy as jnp
from jax import lax
from jax.experimental import pallas as pl
from jax.experimental.pallas import tpu as pltpu
from jax.experimental.pallas import tpu_sc as plsc

_B = 4096
_S = 200
_D = 64
_R = 4
_OUT_S = _R + _S
_PS = 112
_NW = 32
_LANES = 16
_HD = _D // 2


def _make_sc_kernel(nb):
    mesh = plsc.VectorSubcoreMesh(core_axis_name="c", subcore_axis_name="s")
    seq_per_w = nb // _NW

    @functools.partial(
        pl.kernel,
        mesh=mesh,
        compiler_params=pltpu.CompilerParams(use_tc_tiling_on_sc=False),
        out_type=jax.ShapeDtypeStruct((nb, _PS // 2, 2 * _D), jnp.float32),
        scratch_types=[
            pltpu.VMEM((256,), jnp.int32),
            pltpu.VMEM((256,), jnp.int32),
            pltpu.VMEM((_S, _D // 2), jnp.float32),
            pltpu.VMEM((_S, _D // 2), jnp.float32),
            pltpu.SemaphoreType.DMA,
            pltpu.SemaphoreType.DMA,
            pltpu.SemaphoreType.DMA,
            pltpu.SemaphoreType.DMA,
            pltpu.SemaphoreType.DMA,
            pltpu.SemaphoreType.DMA,
        ],
    )
    def emb_kernel(tok_hbm, table_hbm, out_hbm,
                   idx0, idx1, gat0, gat1,
                   gsem0, gsem1, osem0, osem1, isem0, isem1):
        wid = lax.axis_index("s") * 2 + lax.axis_index("c")
        base = wid * seq_per_w

        def start_idx(i, idx, isem):
            pltpu.async_copy(tok_hbm.at[pl.ds((base + i) * 256, 256)], idx,
                             isem)

        def drain_idx(idx, isem):
            pltpu.make_async_copy(tok_hbm.at[pl.ds(0, 256)], idx, isem).wait()

        def start_gather(idx, gat, gsem):
            for k in range(4):
                pltpu.async_copy(
                    table_hbm.at[idx.at[pl.ds(56 * k, _S // 4)]],
                    gat.at[pl.ds((_S // 4) * k, _S // 4)], gsem)

        def drain_gather(gat, gsem):
            pltpu.make_async_copy(table_hbm.at[pl.ds(0, _S)], gat, gsem).wait()

        def start_out(gat, b, osem):
            for k in range(4):
                pltpu.async_copy(
                    gat.at[pl.ds((_S // 4) * k, _S // 4)],
                    out_hbm.at[b, pl.ds(0, _S // 4), pl.ds(_HD * k, _HD)],
                    osem)

        def drain_out(gat, b, osem):
            for k in range(4):
                pltpu.make_async_copy(
                    gat.at[pl.ds((_S // 4) * k, _S // 4)],
                    out_hbm.at[b, pl.ds(0, _S // 4), pl.ds(_HD * k, _HD)],
                    osem).wait()

        start_idx(0, idx0, isem0)
        start_idx(1, idx1, isem1)
        drain_idx(idx0, isem0)
        drain_idx(idx1, isem1)
        start_gather(idx0, gat0, gsem0)
        start_gather(idx1, gat1, gsem1)

        def seq_body(j, carry):
            for s, (idx, gat, gsem, osem, isem) in enumerate((
                    (idx0, gat0, gsem0, osem0, isem0),
                    (idx1, gat1, gsem1, osem1, isem1))):
                i = 2 * j + s
                b = base + i
                drain_gather(gat, gsem)

                @pl.when(i + 2 < seq_per_w)
                def _():
                    start_idx(i + 2, idx, isem)

                start_out(gat, b, osem)

                @pl.when(i + 2 < seq_per_w)
                def _():
                    drain_idx(idx, isem)
                    drain_out(gat, b, osem)
                    start_gather(idx, gat, gsem)
            return carry

        lax.fori_loop(0, seq_per_w // 2, seq_body, 0)
        drain_out(gat0, 0, osem0)
        drain_out(gat1, 0, osem1)

    return emb_kernel


_NHALF = _B // 2
_EMB_KERNEL = _make_sc_kernel(_NHALF)

_BBLK = 128
_HBLKS = _NHALF // _BBLK
_PL = _PS // 2


def _tc_body(x_ref, pos_ref, reg_ref, o_ref):
    w = lax.bitcast_convert_type(x_ref[...], jnp.int32)
    ev = lax.bitcast_convert_type(w << 16, jnp.float32)
    od = lax.bitcast_convert_type(w & jnp.int32(-65536), jnp.float32)
    pos = pos_ref[...]
    for half, part in ((0, ev), (1, od)):
        z = lax.transpose(part, (1, 2, 0))
        z = z.reshape(2 * _PS, _HD, _BBLK)[: _S]
        o_ref[pl.ds(_R, _S), pl.ds(half * _HD, _HD)] = (
            z + pos[:, half * _HD:(half + 1) * _HD][:, :, None])
    o_ref[pl.ds(0, _R)] = jnp.broadcast_to(
        reg_ref[...][:, :, None], (_R, _D, _BBLK))


def _tc_transpose_body2(prev_ref, x_ref, pos_ref, reg_ref, o_ref):
    del prev_ref
    _tc_body(x_ref, pos_ref, reg_ref, o_ref)


_TC_STAGE_H1 = pl.pallas_call(
    _tc_body,
    grid=(_HBLKS,),
    in_specs=[
        pl.BlockSpec((_BBLK, _PL, 2 * _D), lambda i: (i, 0, 0)),
        pl.BlockSpec((_S, _D), lambda i: (0, 0)),
        pl.BlockSpec((_R, _D), lambda i: (0, 0)),
    ],
    out_specs=pl.BlockSpec((_OUT_S, _D, _BBLK), lambda i: (0, 0, i)),
    out_shape=jax.ShapeDtypeStruct((_OUT_S, _D, _B), jnp.float32),
)

_TC_STAGE_H2 = pl.pallas_call(
    _tc_transpose_body2,
    grid=(_HBLKS,),
    in_specs=[
        pl.BlockSpec(memory_space=pl.ANY),
        pl.BlockSpec((_BBLK, _PL, 2 * _D), lambda i: (i, 0, 0)),
        pl.BlockSpec((_S, _D), lambda i: (0, 0)),
        pl.BlockSpec((_R, _D), lambda i: (0, 0)),
    ],
    out_specs=pl.BlockSpec((_OUT_S, _D, _BBLK), lambda i: (0, 0, i + _HBLKS)),
    out_shape=jax.ShapeDtypeStruct((_OUT_S, _D, _B), jnp.float32),
    input_output_aliases={0: 0},
)


@jax.jit
def kernel(tokens, input_embedding, position_embedding, register_tokens):
    v = input_embedding.shape[0]
    tb16 = input_embedding.astype(jnp.bfloat16)
    tperm = tb16.reshape(v, 2, _HD).transpose(0, 2, 1)
    tpack = lax.bitcast_convert_type(tperm, jnp.float32)
    z6 = jnp.zeros((_B, 6), jnp.int32)
    tok_g = jnp.concatenate(
        [tokens[:, 0::4], z6, tokens[:, 1::4], z6, tokens[:, 2::4], z6,
         tokens[:, 3::4], jnp.zeros((_B, 256 - 3 * 56 - _S // 4), jnp.int32)],
        axis=1)
    tok_flat = tok_g.reshape(-1)
    h1 = _EMB_KERNEL(tok_flat[: _NHALF * 256], tpack)
    h2 = _EMB_KERNEL(tok_flat[_NHALF * 256:], tpack)
    s1 = h1
    s2 = h2
    out_t = _TC_STAGE_H1(s1, position_embedding, register_tokens)
    out_t = _TC_STAGE_H2(out_t, s2, position_embedding, register_tokens)
    return out_t.transpose(2, 0, 1)


# speedup vs baseline: 1.9449x; 1.0019x over previous
import functools

import jax
import jax.numpy as jnp
from jax import lax
from jax.experimental import pallas as pl
from jax.experimental.pallas import tpu as pltpu
from jax.experimental.pallas import tpu_sc as plsc

_B = 4096
_S = 200
_D = 64
_R = 4
_OUT_S = _R + _S
_PS = 112
_NW = 32
_LANES = 16
_HD = _D // 2


def _make_sc_kernel(nb):
    mesh = plsc.VectorSubcoreMesh(core_axis_name="c", subcore_axis_name="s")
    seq_per_w = nb // _NW

    @functools.partial(
        pl.kernel,
        mesh=mesh,
        compiler_params=pltpu.CompilerParams(use_tc_tiling_on_sc=False),
        out_type=jax.ShapeDtypeStruct((nb, _PS // 2, 2 * _D), jnp.float32),
        scratch_types=[
            pltpu.VMEM((256,), jnp.int32),
            pltpu.VMEM((256,), jnp.int32),
            pltpu.VMEM((_S, _D // 2), jnp.float32),
            pltpu.VMEM((_S, _D // 2), jnp.float32),
            pltpu.SemaphoreType.DMA,
            pltpu.SemaphoreType.DMA,
            pltpu.SemaphoreType.DMA,
            pltpu.SemaphoreType.DMA,
            pltpu.SemaphoreType.DMA,
            pltpu.SemaphoreType.DMA,
        ],
    )
    def emb_kernel(tok_hbm, table_hbm, out_hbm,
                   idx0, idx1, gat0, gat1,
                   gsem0, gsem1, osem0, osem1, isem0, isem1):
        wid = lax.axis_index("s") * 2 + lax.axis_index("c")
        base = wid * seq_per_w

        def start_idx(i, idx, isem):
            pltpu.async_copy(tok_hbm.at[pl.ds((base + i) * 256, 256)], idx,
                             isem)

        def drain_idx(idx, isem):
            pltpu.make_async_copy(tok_hbm.at[pl.ds(0, 256)], idx, isem).wait()

        def start_gather(idx, gat, gsem):
            for k in range(4):
                pltpu.async_copy(
                    table_hbm.at[idx.at[pl.ds(56 * k, _S // 4)]],
                    gat.at[pl.ds((_S // 4) * k, _S // 4)], gsem)

        def drain_gather(gat, gsem):
            pltpu.make_async_copy(table_hbm.at[pl.ds(0, _S)], gat, gsem).wait()

        def start_out(gat, b, osem):
            for k in range(4):
                pltpu.async_copy(
                    gat.at[pl.ds((_S // 4) * k, _S // 4)],
                    out_hbm.at[b, pl.ds(0, _S // 4), pl.ds(_HD * k, _HD)],
                    osem)

        def drain_out(gat, b, osem):
            for k in range(4):
                pltpu.make_async_copy(
                    gat.at[pl.ds((_S // 4) * k, _S // 4)],
                    out_hbm.at[b, pl.ds(0, _S // 4), pl.ds(_HD * k, _HD)],
                    osem).wait()

        start_idx(0, idx0, isem0)
        start_idx(1, idx1, isem1)
        drain_idx(idx0, isem0)
        drain_idx(idx1, isem1)
        start_gather(idx0, gat0, gsem0)
        start_gather(idx1, gat1, gsem1)

        def seq_body(j, carry):
            for s, (idx, gat, gsem, osem, isem) in enumerate((
                    (idx0, gat0, gsem0, osem0, isem0),
                    (idx1, gat1, gsem1, osem1, isem1))):
                i = 2 * j + s
                b = base + i
                drain_gather(gat, gsem)

                @pl.when(i + 2 < seq_per_w)
                def _():
                    start_idx(i + 2, idx, isem)

                start_out(gat, b, osem)

                @pl.when(i + 2 < seq_per_w)
                def _():
                    drain_idx(idx, isem)
                    drain_out(gat, b, osem)
                    start_gather(idx, gat, gsem)
            return carry

        lax.fori_loop(0, seq_per_w // 2, seq_body, 0)
        drain_out(gat0, 0, osem0)
        drain_out(gat1, 0, osem1)

    return emb_kernel


_NHALF = _B // 2
_EMB_KERNEL = _make_sc_kernel(_NHALF)

_BBLK = 128
_HBLKS = _NHALF // _BBLK
_PL = _PS // 2


def _tc_body(x_ref, pos_ref, reg_ref, o_ref):
    w = lax.bitcast_convert_type(x_ref[...], jnp.int32)
    ev = lax.bitcast_convert_type(w << 16, jnp.float32)
    od = lax.bitcast_convert_type(w & jnp.int32(-65536), jnp.float32)
    ze = lax.transpose(ev, (1, 2, 0)).reshape(2 * _PS, _HD, _BBLK)[: _S]
    zo = lax.transpose(od, (1, 2, 0)).reshape(2 * _PS, _HD, _BBLK)[: _S]
    z = jnp.concatenate((ze, zo), axis=1)
    o_ref[pl.ds(_R, _S)] = z + pos_ref[...][:, :, None]
    o_ref[pl.ds(0, _R)] = jnp.broadcast_to(
        reg_ref[...][:, :, None], (_R, _D, _BBLK))


def _tc_transpose_body2(prev_ref, x_ref, pos_ref, reg_ref, o_ref):
    del prev_ref
    _tc_body(x_ref, pos_ref, reg_ref, o_ref)


_TC_STAGE_H1 = pl.pallas_call(
    _tc_body,
    grid=(_HBLKS,),
    in_specs=[
        pl.BlockSpec((_BBLK, _PL, 2 * _D), lambda i: (i, 0, 0)),
        pl.BlockSpec((_S, _D), lambda i: (0, 0)),
        pl.BlockSpec((_R, _D), lambda i: (0, 0)),
    ],
    out_specs=pl.BlockSpec((_OUT_S, _D, _BBLK), lambda i: (0, 0, i)),
    out_shape=jax.ShapeDtypeStruct((_OUT_S, _D, _B), jnp.float32),
)

_TC_STAGE_H2 = pl.pallas_call(
    _tc_transpose_body2,
    grid=(_HBLKS,),
    in_specs=[
        pl.BlockSpec(memory_space=pl.ANY),
        pl.BlockSpec((_BBLK, _PL, 2 * _D), lambda i: (i, 0, 0)),
        pl.BlockSpec((_S, _D), lambda i: (0, 0)),
        pl.BlockSpec((_R, _D), lambda i: (0, 0)),
    ],
    out_specs=pl.BlockSpec((_OUT_S, _D, _BBLK), lambda i: (0, 0, i + _HBLKS)),
    out_shape=jax.ShapeDtypeStruct((_OUT_S, _D, _B), jnp.float32),
    input_output_aliases={0: 0},
)


@jax.jit
def kernel(tokens, input_embedding, position_embedding, register_tokens):
    v = input_embedding.shape[0]
    tb16 = input_embedding.astype(jnp.bfloat16)
    tperm = tb16.reshape(v, 2, _HD).transpose(0, 2, 1)
    tpack = lax.bitcast_convert_type(tperm, jnp.float32)
    z6 = jnp.zeros((_B, 6), jnp.int32)
    tok_g = jnp.concatenate(
        [tokens[:, 0::4], z6, tokens[:, 1::4], z6, tokens[:, 2::4], z6,
         tokens[:, 3::4], jnp.zeros((_B, 256 - 3 * 56 - _S // 4), jnp.int32)],
        axis=1)
    tok_flat = tok_g.reshape(-1)
    h1 = _EMB_KERNEL(tok_flat[: _NHALF * 256], tpack)
    h2 = _EMB_KERNEL(tok_flat[_NHALF * 256:], tpack)
    s1 = h1
    s2 = h2
    out_t = _TC_STAGE_H1(s1, position_embedding, register_tokens)
    out_t = _TC_STAGE_H2(out_t, s2, position_embedding, register_tokens)
    return out_t.transpose(2, 0, 1)


# speedup vs baseline: 2.3092x; 1.1873x over previous
import functools

import jax
import jax.numpy as jnp
from jax import lax
from jax.experimental import pallas as pl
from jax.experimental.pallas import tpu as pltpu
from jax.experimental.pallas import tpu_sc as plsc

_B = 4096
_S = 200
_D = 64
_R = 4
_OUT_S = _R + _S
_PS = 112
_NW = 32
_LANES = 16
_HD = _D // 2


def _make_sc_kernel(nb):
    mesh = plsc.VectorSubcoreMesh(core_axis_name="c", subcore_axis_name="s")
    seq_per_w = nb // _NW

    @functools.partial(
        pl.kernel,
        mesh=mesh,
        compiler_params=pltpu.CompilerParams(use_tc_tiling_on_sc=False),
        out_type=jax.ShapeDtypeStruct((nb, _PS // 2, 2 * _D), jnp.float32),
        scratch_types=[
            pltpu.VMEM((256,), jnp.int32),
            pltpu.VMEM((256,), jnp.int32),
            pltpu.VMEM((_S, _D // 2), jnp.float32),
            pltpu.VMEM((_S, _D // 2), jnp.float32),
            pltpu.SemaphoreType.DMA,
            pltpu.SemaphoreType.DMA,
            pltpu.SemaphoreType.DMA,
            pltpu.SemaphoreType.DMA,
            pltpu.SemaphoreType.DMA,
            pltpu.SemaphoreType.DMA,
        ],
    )
    def emb_kernel(tok_hbm, table_hbm, out_hbm,
                   idx0, idx1, gat0, gat1,
                   gsem0, gsem1, osem0, osem1, isem0, isem1):
        wid = lax.axis_index("s") * 2 + lax.axis_index("c")
        base = wid * seq_per_w

        def start_idx(i, idx, isem):
            pltpu.async_copy(tok_hbm.at[pl.ds((base + i) * 256, 256)], idx,
                             isem)

        def drain_idx(idx, isem):
            pltpu.make_async_copy(tok_hbm.at[pl.ds(0, 256)], idx, isem).wait()

        def start_gather(idx, gat, gsem):
            for k in range(4):
                pltpu.async_copy(
                    table_hbm.at[idx.at[pl.ds(56 * k, _S // 4)]],
                    gat.at[pl.ds((_S // 4) * k, _S // 4)], gsem)

        def drain_gather(gat, gsem):
            pltpu.make_async_copy(table_hbm.at[pl.ds(0, _S)], gat, gsem).wait()

        def start_out(gat, b, osem):
            for k in range(4):
                pltpu.async_copy(
                    gat.at[pl.ds((_S // 4) * k, _S // 4)],
                    out_hbm.at[b, pl.ds(0, _S // 4), pl.ds(_HD * k, _HD)],
                    osem)

        def drain_out(gat, b, osem):
            for k in range(4):
                pltpu.make_async_copy(
                    gat.at[pl.ds((_S // 4) * k, _S // 4)],
                    out_hbm.at[b, pl.ds(0, _S // 4), pl.ds(_HD * k, _HD)],
                    osem).wait()

        start_idx(0, idx0, isem0)
        start_idx(1, idx1, isem1)
        drain_idx(idx0, isem0)
        drain_idx(idx1, isem1)
        start_gather(idx0, gat0, gsem0)
        start_gather(idx1, gat1, gsem1)

        def seq_body(j, carry):
            for s, (idx, gat, gsem, osem, isem) in enumerate((
                    (idx0, gat0, gsem0, osem0, isem0),
                    (idx1, gat1, gsem1, osem1, isem1))):
                i = 2 * j + s
                b = base + i
                drain_gather(gat, gsem)

                @pl.when(i + 2 < seq_per_w)
                def _():
                    start_idx(i + 2, idx, isem)

                start_out(gat, b, osem)

                @pl.when(i + 2 < seq_per_w)
                def _():
                    drain_idx(idx, isem)
                    drain_out(gat, b, osem)
                    start_gather(idx, gat, gsem)
            return carry

        lax.fori_loop(0, seq_per_w // 2, seq_body, 0)
        drain_out(gat0, 0, osem0)
        drain_out(gat1, 0, osem1)

    return emb_kernel


_NHALF = _B // 2
_EMB_KERNEL = _make_sc_kernel(_NHALF)

_BBLK = 128
_HBLKS = _NHALF // _BBLK
_PL = _PS // 2


def _tc_body(x_ref, pos_ref, reg_ref, o_ref):
    w = lax.bitcast_convert_type(x_ref[...], jnp.int32)
    zw = lax.transpose(w, (1, 2, 0)).reshape(2 * _PS, _HD, _BBLK)[: _S]
    ze = lax.bitcast_convert_type(zw << 16, jnp.float32)
    zo = lax.bitcast_convert_type(zw & jnp.int32(-65536), jnp.float32)
    z = jnp.concatenate((ze, zo), axis=1)
    o_ref[pl.ds(_R, _S)] = z + pos_ref[...][:, :, None]
    o_ref[pl.ds(0, _R)] = jnp.broadcast_to(
        reg_ref[...][:, :, None], (_R, _D, _BBLK))


def _tc_transpose_body2(prev_ref, x_ref, pos_ref, reg_ref, o_ref):
    del prev_ref
    _tc_body(x_ref, pos_ref, reg_ref, o_ref)


_TC_STAGE_H1 = pl.pallas_call(
    _tc_body,
    grid=(_HBLKS,),
    in_specs=[
        pl.BlockSpec((_BBLK, _PL, 2 * _D), lambda i: (i, 0, 0)),
        pl.BlockSpec((_S, _D), lambda i: (0, 0)),
        pl.BlockSpec((_R, _D), lambda i: (0, 0)),
    ],
    out_specs=pl.BlockSpec((_OUT_S, _D, _BBLK), lambda i: (0, 0, i)),
    out_shape=jax.ShapeDtypeStruct((_OUT_S, _D, _B), jnp.float32),
)

_TC_STAGE_H2 = pl.pallas_call(
    _tc_transpose_body2,
    grid=(_HBLKS,),
    in_specs=[
        pl.BlockSpec(memory_space=pl.ANY),
        pl.BlockSpec((_BBLK, _PL, 2 * _D), lambda i: (i, 0, 0)),
        pl.BlockSpec((_S, _D), lambda i: (0, 0)),
        pl.BlockSpec((_R, _D), lambda i: (0, 0)),
    ],
    out_specs=pl.BlockSpec((_OUT_S, _D, _BBLK), lambda i: (0, 0, i + _HBLKS)),
    out_shape=jax.ShapeDtypeStruct((_OUT_S, _D, _B), jnp.float32),
    input_output_aliases={0: 0},
)


@jax.jit
def kernel(tokens, input_embedding, position_embedding, register_tokens):
    v = input_embedding.shape[0]
    tb16 = input_embedding.astype(jnp.bfloat16)
    tperm = tb16.reshape(v, 2, _HD).transpose(0, 2, 1)
    tpack = lax.bitcast_convert_type(tperm, jnp.float32)
    z6 = jnp.zeros((_B, 6), jnp.int32)
    tok_g = jnp.concatenate(
        [tokens[:, 0::4], z6, tokens[:, 1::4], z6, tokens[:, 2::4], z6,
         tokens[:, 3::4], jnp.zeros((_B, 256 - 3 * 56 - _S // 4), jnp.int32)],
        axis=1)
    tok_flat = tok_g.reshape(-1)
    h1 = _EMB_KERNEL(tok_flat[: _NHALF * 256], tpack)
    h2 = _EMB_KERNEL(tok_flat[_NHALF * 256:], tpack)
    s1 = h1
    s2 = h2
    out_t = _TC_STAGE_H1(s1, position_embedding, register_tokens)
    out_t = _TC_STAGE_H2(out_t, s2, position_embedding, register_tokens)
    return out_t.transpose(2, 0, 1)


# speedup vs baseline: 2.3575x; 1.0209x over previous
import functools

import jax
import jax.numpy as jnp
from jax import lax
from jax.experimental import pallas as pl
from jax.experimental.pallas import tpu as pltpu
from jax.experimental.pallas import tpu_sc as plsc

_B = 4096
_S = 200
_D = 64
_R = 4
_OUT_S = _R + _S
_PS = 104
_NW = 32
_SEQ_PER_W = _B // _NW
_LANES = 16
_C0 = 128
_C1 = _S - _C0


def _make_sc_kernel(nb):
    mesh = plsc.VectorSubcoreMesh(core_axis_name="c", subcore_axis_name="s")
    seq_per_w = nb // _NW

    @functools.partial(
        pl.kernel,
        mesh=mesh,
        compiler_params=pltpu.CompilerParams(use_tc_tiling_on_sc=False),
        out_type=jax.ShapeDtypeStruct((nb, _PS, 2 * _D), jnp.float32),
        scratch_types=[
            pltpu.VMEM((_S // 2, 2 * _D), jnp.float32),
            pltpu.VMEM((256,), jnp.int32),
            pltpu.VMEM((256,), jnp.int32),
            pltpu.VMEM((_S, _D), jnp.float32),
            pltpu.VMEM((_S, _D), jnp.float32),
            pltpu.VMEM((_PS, 2 * _D), jnp.float32),
            pltpu.VMEM((_PS, 2 * _D), jnp.float32),
            pltpu.SemaphoreType.DMA,
            pltpu.SemaphoreType.DMA,
            pltpu.SemaphoreType.DMA,
            pltpu.SemaphoreType.DMA,
            pltpu.SemaphoreType.DMA,
            pltpu.SemaphoreType.DMA,
        ],
    )
    def emb_kernel(tok_hbm, table_hbm, pos_hbm, reg_hbm, out_hbm,
                   pos_v, idx0, idx1, gat0, gat1, buf0, buf1,
                   gsem0, gsem1, osem0, osem1, isem0, isem1):
        wid = lax.axis_index("s") * 2 + lax.axis_index("c")
        base = wid * seq_per_w
        pltpu.sync_copy(pos_hbm, pos_v)
        pltpu.sync_copy(reg_hbm, buf0.at[pl.ds(0, _R // 2)])
        pltpu.sync_copy(reg_hbm, buf1.at[pl.ds(0, _R // 2)])

        def start_idx(i, idx, isem):
            pltpu.async_copy(tok_hbm.at[pl.ds((base + i) * 256, 256)], idx,
                             isem)

        def drain_idx(idx, isem):
            pltpu.make_async_copy(tok_hbm.at[pl.ds(0, 256)], idx, isem).wait()

        def start_gather(idx, gat, gsem):
            pltpu.async_copy(table_hbm.at[idx.at[pl.ds(0, _C0)]],
                             gat.at[pl.ds(0, _C0)], gsem)
            pltpu.async_copy(table_hbm.at[idx.at[pl.ds(_C0, _C1)]],
                             gat.at[pl.ds(_C0, _C1)], gsem)

        def drain_gather(gat, gsem):
            pltpu.make_async_copy(table_hbm.at[pl.ds(0, _S)], gat, gsem).wait()

        start_idx(0, idx0, isem0)
        start_idx(1, idx1, isem1)
        drain_idx(idx0, isem0)
        drain_idx(idx1, isem1)
        start_gather(idx0, gat0, gsem0)
        start_gather(idx1, gat1, gsem1)

        def seq_body(j, carry):
            for s, (idx, gat, buf, gsem, osem, isem) in enumerate((
                    (idx0, gat0, buf0, gsem0, osem0, isem0),
                    (idx1, gat1, buf1, gsem1, osem1, isem1))):
                i = 2 * j + s
                b = base + i
                drain_gather(gat, gsem)

                @pl.when(i + 2 < seq_per_w)
                def _():
                    start_idx(i + 2, idx, isem)

                @pl.when(j > 0)
                def _():
                    pltpu.make_async_copy(buf, out_hbm.at[b], osem).wait()

                def add_rows(r2, c2):
                    for c in range(_D // _LANES):
                        sl = pl.ds(c * _LANES, _LANES)
                        buf[r2 + _R // 2, sl] = (
                            gat[2 * r2, sl]
                            + pos_v[r2, pl.ds(c * _LANES, _LANES)])
                        buf[r2 + _R // 2, pl.ds(_D + c * _LANES, _LANES)] = (
                            gat[2 * r2 + 1, sl]
                            + pos_v[r2, pl.ds(_D + c * _LANES, _LANES)])
                    return c2

                lax.fori_loop(0, _S // 2, add_rows, 0)
                pltpu.async_copy(buf, out_hbm.at[b], osem)

                @pl.when(i + 2 < seq_per_w)
                def _():
                    drain_idx(idx, isem)
                    start_gather(idx, gat, gsem)
            return carry

        lax.fori_loop(0, seq_per_w // 2, seq_body, 0)
        pltpu.make_async_copy(buf0, out_hbm.at[0], osem0).wait()
        pltpu.make_async_copy(buf1, out_hbm.at[0], osem1).wait()

    return emb_kernel


_NCHUNK = 4
_NSUB = _B // _NCHUNK
_EMB_KERNEL = _make_sc_kernel(_NSUB)

_BBLK = 128
_CBLKS = _NSUB // _BBLK


def _tc_transpose_body(x_ref, o_ref):
    x = x_ref[...]
    z = lax.transpose(x, (1, 2, 0))
    z = z.reshape(_PS * 2 * _D, _BBLK)
    o_ref[...] = z[: _OUT_S * _D].reshape(_OUT_S, _D, _BBLK)


def _tc_transpose_body2(prev_ref, x_ref, o_ref):
    del prev_ref
    _tc_transpose_body(x_ref, o_ref)


def _make_tc_stage(chunk):
    off = chunk * _CBLKS
    if chunk == 0:
        return pl.pallas_call(
            _tc_transpose_body,
            grid=(_CBLKS,),
            in_specs=[pl.BlockSpec((_BBLK, _PS, 2 * _D), lambda i: (i, 0, 0))],
            out_specs=pl.BlockSpec((_OUT_S, _D, _BBLK), lambda i: (0, 0, i)),
            out_shape=jax.ShapeDtypeStruct((_OUT_S, _D, _B), jnp.float32),
        )
    return pl.pallas_call(
        _tc_transpose_body2,
        grid=(_CBLKS,),
        in_specs=[
            pl.BlockSpec(memory_space=pl.ANY),
            pl.BlockSpec((_BBLK, _PS, 2 * _D), lambda i: (i, 0, 0)),
        ],
        out_specs=pl.BlockSpec((_OUT_S, _D, _BBLK),
                               lambda i: (0, 0, i + off)),
        out_shape=jax.ShapeDtypeStruct((_OUT_S, _D, _B), jnp.float32),
        input_output_aliases={0: 0},
    )


_TC_STAGES = [_make_tc_stage(c) for c in range(_NCHUNK)]


@jax.jit
def kernel(tokens, input_embedding, position_embedding, register_tokens):
    pos2 = position_embedding.reshape(_S // 2, 2 * _D)
    reg2 = register_tokens.reshape(_R // 2, 2 * _D)
    tok_flat = jnp.pad(tokens, ((0, 0), (0, 256 - _S))).reshape(-1)
    hs = [_EMB_KERNEL(tok_flat[c * _NSUB * 256:(c + 1) * _NSUB * 256],
                      input_embedding, pos2, reg2)
          for c in range(_NCHUNK)]
    out_t = _TC_STAGES[0](hs[0])
    for c in range(1, _NCHUNK):
        out_t = _TC_STAGES[c](out_t, hs[c])
    return out_t.transpose(2, 0, 1)


# speedup vs baseline: 2.3680x; 1.0045x over previous
import functools

import jax
import jax.numpy as jnp
from jax import lax
from jax.experimental import pallas as pl
from jax.experimental.pallas import tpu as pltpu
from jax.experimental.pallas import tpu_sc as plsc

_B = 4096
_S = 200
_D = 64
_R = 4
_OUT_S = _R + _S
_PS = 104
_NW = 32
_SEQ_PER_W = _B // _NW
_LANES = 16
_C0 = 128
_C1 = _S - _C0


def _make_sc_kernel(nb):
    mesh = plsc.VectorSubcoreMesh(core_axis_name="c", subcore_axis_name="s")
    seq_per_w = nb // _NW

    @functools.partial(
        pl.kernel,
        mesh=mesh,
        compiler_params=pltpu.CompilerParams(use_tc_tiling_on_sc=False),
        out_type=jax.ShapeDtypeStruct((nb, _PS, 2 * _D), jnp.float32),
        scratch_types=[
            pltpu.VMEM((_S // 2, 2 * _D), jnp.float32),
            pltpu.VMEM((256,), jnp.int32),
            pltpu.VMEM((256,), jnp.int32),
            pltpu.VMEM((_S, _D), jnp.float32),
            pltpu.VMEM((_S, _D), jnp.float32),
            pltpu.VMEM((_PS, 2 * _D), jnp.float32),
            pltpu.VMEM((_PS, 2 * _D), jnp.float32),
            pltpu.SemaphoreType.DMA,
            pltpu.SemaphoreType.DMA,
            pltpu.SemaphoreType.DMA,
            pltpu.SemaphoreType.DMA,
            pltpu.SemaphoreType.DMA,
            pltpu.SemaphoreType.DMA,
        ],
    )
    def emb_kernel(tok_hbm, table_hbm, pos_hbm, reg_hbm, out_hbm,
                   pos_v, idx0, idx1, gat0, gat1, buf0, buf1,
                   gsem0, gsem1, osem0, osem1, isem0, isem1):
        wid = lax.axis_index("s") * 2 + lax.axis_index("c")
        base = wid * seq_per_w
        pltpu.sync_copy(pos_hbm, pos_v)
        pltpu.sync_copy(reg_hbm, buf0.at[pl.ds(0, _R // 2)])
        pltpu.sync_copy(reg_hbm, buf1.at[pl.ds(0, _R // 2)])

        def start_idx(i, idx, isem):
            pltpu.async_copy(tok_hbm.at[pl.ds((base + i) * 256, 256)], idx,
                             isem)

        def drain_idx(idx, isem):
            pltpu.make_async_copy(tok_hbm.at[pl.ds(0, 256)], idx, isem).wait()

        def start_gather(idx, gat, gsem):
            pltpu.async_copy(table_hbm.at[idx.at[pl.ds(0, _C0)]],
                             gat.at[pl.ds(0, _C0)], gsem)
            pltpu.async_copy(table_hbm.at[idx.at[pl.ds(_C0, _C1)]],
                             gat.at[pl.ds(_C0, _C1)], gsem)

        def drain_gather(gat, gsem):
            pltpu.make_async_copy(table_hbm.at[pl.ds(0, _S)], gat, gsem).wait()

        start_idx(0, idx0, isem0)
        start_idx(1, idx1, isem1)
        drain_idx(idx0, isem0)
        drain_idx(idx1, isem1)
        start_gather(idx0, gat0, gsem0)
        start_gather(idx1, gat1, gsem1)

        def seq_body(j, carry):
            for s, (idx, gat, buf, gsem, osem, isem) in enumerate((
                    (idx0, gat0, buf0, gsem0, osem0, isem0),
                    (idx1, gat1, buf1, gsem1, osem1, isem1))):
                i = 2 * j + s
                b = base + i
                drain_gather(gat, gsem)

                @pl.when(i + 2 < seq_per_w)
                def _():
                    start_idx(i + 2, idx, isem)

                @pl.when(j > 0)
                def _():
                    pltpu.make_async_copy(buf, out_hbm.at[b], osem).wait()

                def add_rows(r2, c2):
                    for c in range(_D // _LANES):
                        sl = pl.ds(c * _LANES, _LANES)
                        buf[r2 + _R // 2, sl] = (
                            gat[2 * r2, sl]
                            + pos_v[r2, pl.ds(c * _LANES, _LANES)])
                        buf[r2 + _R // 2, pl.ds(_D + c * _LANES, _LANES)] = (
                            gat[2 * r2 + 1, sl]
                            + pos_v[r2, pl.ds(_D + c * _LANES, _LANES)])
                    return c2

                lax.fori_loop(0, _S // 2, add_rows, 0)
                pltpu.async_copy(buf, out_hbm.at[b], osem)

                @pl.when(i + 2 < seq_per_w)
                def _():
                    drain_idx(idx, isem)
                    start_gather(idx, gat, gsem)
            return carry

        lax.fori_loop(0, seq_per_w // 2, seq_body, 0)
        pltpu.make_async_copy(buf0, out_hbm.at[0], osem0).wait()
        pltpu.make_async_copy(buf1, out_hbm.at[0], osem1).wait()

    return emb_kernel


_NHALF = _B // 2
_EMB_KERNEL = _make_sc_kernel(_NHALF)

_BBLK = 128
_HBLKS = _NHALF // _BBLK


def _tc_transpose_body(x_ref, o_ref):
    x = x_ref[...]
    z = lax.transpose(x, (1, 2, 0))
    z = z.reshape(_PS * 2 * _D, _BBLK)
    o_ref[...] = z[: _OUT_S * _D].reshape(_OUT_S, _D, _BBLK)


def _tc_transpose_body2(prev_ref, x_ref, o_ref):
    del prev_ref
    _tc_transpose_body(x_ref, o_ref)


_TC_TRANSPOSE_H1 = pl.pallas_call(
    _tc_transpose_body,
    grid=(_HBLKS,),
    in_specs=[pl.BlockSpec((_BBLK, _PS, 2 * _D), lambda i: (i, 0, 0))],
    out_specs=pl.BlockSpec((_OUT_S, _D, _BBLK), lambda i: (0, 0, i)),
    out_shape=jax.ShapeDtypeStruct((_OUT_S, _D, _B), jnp.float32),
)

_TC_TRANSPOSE_H2 = pl.pallas_call(
    _tc_transpose_body2,
    grid=(_HBLKS,),
    in_specs=[
        pl.BlockSpec(memory_space=pl.ANY),
        pl.BlockSpec((_BBLK, _PS, 2 * _D), lambda i: (i, 0, 0)),
    ],
    out_specs=pl.BlockSpec((_OUT_S, _D, _BBLK), lambda i: (0, 0, i + _HBLKS)),
    out_shape=jax.ShapeDtypeStruct((_OUT_S, _D, _B), jnp.float32),
    input_output_aliases={0: 0},
)


@jax.jit
def kernel(tokens, input_embedding, position_embedding, register_tokens):
    pos2 = position_embedding.reshape(_S // 2, 2 * _D)
    reg2 = register_tokens.reshape(_R // 2, 2 * _D)
    tok_flat = jnp.pad(tokens, ((0, 0), (0, 256 - _S))).reshape(-1)
    h1 = _EMB_KERNEL(tok_flat[: _NHALF * 256], input_embedding, pos2, reg2)
    h2 = _EMB_KERNEL(tok_flat[_NHALF * 256:], input_embedding, pos2, reg2)
    out_t = _TC_TRANSPOSE_H1(h1)
    out_t = _TC_TRANSPOSE_H2(out_t, h2)
    return out_t.transpose(2, 0, 1)
